# baseline (speedup 1.0000x reference)
import functools

import jax
import jax.numpy as jnp
from jax import lax
from jax.experimental import pallas as pl
from jax.experimental.pallas import tpu as pltpu
from jax.experimental.pallas import tpu_sc as plsc

_EPS = 1e-7
_LANES = 16


def _sc_scores_kernel(idx_hbm, w_hbm, out_hbm, inv_hbm, idx_v, sc_v, wtab_v,
                      etab_v, inv_v, *,
                      n_cores, samples_per_worker, seq_len, vocab, nlex):
    chunk = samples_per_worker * seq_len
    n_groups = samples_per_worker // _LANES

    wid = lax.axis_index("s") * n_cores + lax.axis_index("c")
    base = wid * chunk

    pltpu.sync_copy(idx_hbm.at[pl.ds(base, chunk)], idx_v)
    pltpu.sync_copy(w_hbm, wtab_v)

    lanes = lax.broadcasted_iota(jnp.int32, (_LANES,), 0)

    for i in range(vocab // _LANES):
        flat = (lanes + i * _LANES) * nlex
        s = plsc.load_gather(wtab_v, [flat])
        for n in range(1, nlex):
            s = s + plsc.load_gather(wtab_v, [flat + n])
        e2 = jnp.exp(-2.0 * jnp.abs(s))
        th = (1.0 - e2) / (1.0 + e2)
        th = jnp.where(s < 0.0, -th, th)
        etab_v[pl.ds(i * _LANES, _LANES)] = jnp.exp(th)

    unroll = 8
    for g in range(n_groups):
        sbase = (lanes + g * _LANES) * seq_len

        def body(j, acc, sbase=sbase, g=g):
            l0 = j * unroll
            for u in range(unroll):
                iv = plsc.load_gather(idx_v, [sbase + (l0 + u)])
                ev = plsc.load_gather(etab_v, [iv])
                sc_v[pl.ds((l0 + u) * samples_per_worker + g * _LANES,
                           _LANES)] = ev
                acc = acc + ev
            return acc

        acc = lax.fori_loop(0, seq_len // unroll, body,
                            jnp.zeros((_LANES,), jnp.float32))
        inv_v[pl.ds(g * _LANES, _LANES)] = 1.0 / (acc + _EPS)

    pltpu.sync_copy(sc_v, out_hbm.at[pl.ds(base, chunk)])
    pltpu.sync_copy(inv_v, inv_hbm.at[pl.ds(wid * samples_per_worker,
                                            samples_per_worker)])


def _tc_pool_kernel(a_ref, i_ref, x_ref, o_ref, *, n_steps):
    j = pl.program_id(0)
    av = a_ref[...]
    xv = x_ref[...]
    part = jnp.sum(xv * jnp.broadcast_to(av, xv.shape), axis=0)

    @pl.when(j == 0)
    def _init():
        o_ref[...] = part

    @pl.when(j != 0)
    def _acc():
        o_ref[...] = o_ref[...] + part

    @pl.when(j == n_steps - 1)
    def _norm():
        o_ref[...] = o_ref[...] * jnp.broadcast_to(i_ref[...], o_ref.shape)


def kernel(lex_indices, x, W):
    B, L = lex_indices.shape
    _, _, D = x.shape
    V, NLEX = W.shape

    info = plsc.get_sparse_core_info()
    nc, ns = info.num_cores, info.num_subcores
    nw = nc * ns
    spw = B // nw
    chunk = spw * L

    mesh = plsc.VectorSubcoreMesh(core_axis_name="c", subcore_axis_name="s")
    sc_scores = pl.kernel(
        functools.partial(
            _sc_scores_kernel,
            n_cores=nc, samples_per_worker=spw,
            seq_len=L, vocab=V, nlex=NLEX),
        out_type=(jax.ShapeDtypeStruct((B * L,), jnp.float32),
                  jax.ShapeDtypeStruct((B,), jnp.float32)),
        mesh=mesh,
        compiler_params=pltpu.CompilerParams(needs_layout_passes=False),
        scratch_types=[
            pltpu.VMEM((chunk,), jnp.int32),
            pltpu.VMEM((chunk,), jnp.float32),
            pltpu.VMEM((V * NLEX,), jnp.float32),
            pltpu.VMEM((V,), jnp.float32),
            pltpu.VMEM((spw,), jnp.float32),
        ],
    )
    a, inv = sc_scores(lex_indices.reshape(B * L), W.reshape(V * NLEX))

    at = a.reshape(nw, L, spw).transpose(1, 0, 2).reshape(L, 1, B)
    inv2 = inv.reshape(1, B)
    xt = x.transpose(1, 2, 0)

    LB = 8
    n_steps = L // LB
    out_t = pl.pallas_call(
        functools.partial(_tc_pool_kernel, n_steps=n_steps),
        grid=(n_steps,),
        in_specs=[
            pl.BlockSpec((LB, 1, B), lambda i: (i, 0, 0)),
            pl.BlockSpec((1, B), lambda i: (0, 0)),
            pl.BlockSpec((LB, D, B), lambda i: (i, 0, 0)),
        ],
        out_specs=pl.BlockSpec((D, B), lambda i: (0, 0)),
        out_shape=jax.ShapeDtypeStruct((D, B), jnp.float32),
    )(at, inv2, xt)
    return out_t.T


# speedup vs baseline: 1.0533x; 1.0533x over previous
import functools

import jax
import jax.numpy as jnp
from jax import lax
from jax.experimental import pallas as pl
from jax.experimental.pallas import tpu as pltpu
from jax.experimental.pallas import tpu_sc as plsc

_EPS = 1e-7
_LANES = 16


def _sc_scores_kernel(idx_hbm, w_hbm, out_hbm, inv_hbm, idx_v, sc_v, wtab_v,
                      etab_v, inv_v, *,
                      n_cores, samples_per_worker, seq_len, vocab, nlex):
    chunk = samples_per_worker * seq_len
    n_groups = samples_per_worker // _LANES

    wid = lax.axis_index("s") * n_cores + lax.axis_index("c")
    base = wid * chunk

    pltpu.sync_copy(idx_hbm.at[pl.ds(base, chunk)], idx_v)
    pltpu.sync_copy(w_hbm, wtab_v)

    lanes = lax.broadcasted_iota(jnp.int32, (_LANES,), 0)

    for i in range(vocab // _LANES):
        flat = (lanes + i * _LANES) * nlex
        s = plsc.load_gather(wtab_v, [flat])
        for n in range(1, nlex):
            s = s + plsc.load_gather(wtab_v, [flat + n])
        e2 = jnp.exp(-2.0 * jnp.abs(s))
        th = (1.0 - e2) / (1.0 + e2)
        th = jnp.where(s < 0.0, -th, th)
        etab_v[pl.ds(i * _LANES, _LANES)] = jnp.exp(th)

    unroll = 8
    for g in range(n_groups):
        sbase = (lanes + g * _LANES) * seq_len

        def body(j, acc, sbase=sbase, g=g):
            l0 = j * unroll
            for u in range(unroll):
                iv = plsc.load_gather(idx_v, [sbase + (l0 + u)])
                ev = plsc.load_gather(etab_v, [iv])
                sc_v[pl.ds((l0 + u) * samples_per_worker + g * _LANES,
                           _LANES)] = ev
                acc = acc + ev
            return acc

        acc = lax.fori_loop(0, seq_len // unroll, body,
                            jnp.zeros((_LANES,), jnp.float32))
        inv_v[pl.ds(g * _LANES, _LANES)] = 1.0 / (acc + _EPS)

    pltpu.sync_copy(sc_v, out_hbm.at[pl.ds(base, chunk)])
    pltpu.sync_copy(inv_v, inv_hbm.at[pl.ds(wid * samples_per_worker,
                                            samples_per_worker)])


def _tc_pool_kernel(a_ref, i_ref, x_ref, o_ref):
    av = a_ref[0]
    xv = x_ref[...]
    s = jnp.sum(xv * jnp.broadcast_to(av, xv.shape), axis=0)
    o_ref[...] = s * jnp.broadcast_to(i_ref[0], s.shape)


def kernel(lex_indices, x, W):
    B, L = lex_indices.shape
    _, _, D = x.shape
    V, NLEX = W.shape

    info = plsc.get_sparse_core_info()
    nc, ns = info.num_cores, info.num_subcores
    nw = nc * ns
    spw = B // nw
    chunk = spw * L

    mesh = plsc.VectorSubcoreMesh(core_axis_name="c", subcore_axis_name="s")
    sc_scores = pl.kernel(
        functools.partial(
            _sc_scores_kernel,
            n_cores=nc, samples_per_worker=spw,
            seq_len=L, vocab=V, nlex=NLEX),
        out_type=(jax.ShapeDtypeStruct((B * L,), jnp.float32),
                  jax.ShapeDtypeStruct((B,), jnp.float32)),
        mesh=mesh,
        compiler_params=pltpu.CompilerParams(needs_layout_passes=False),
        scratch_types=[
            pltpu.VMEM((chunk,), jnp.int32),
            pltpu.VMEM((chunk,), jnp.float32),
            pltpu.VMEM((V * NLEX,), jnp.float32),
            pltpu.VMEM((V,), jnp.float32),
            pltpu.VMEM((spw,), jnp.float32),
        ],
    )
    a, inv = sc_scores(lex_indices.reshape(B * L), W.reshape(V * NLEX))

    BW = spw
    a4 = a.reshape(nw, L, 1, BW)
    inv3 = inv.reshape(nw, 1, BW)
    xt = x.transpose(1, 2, 0)
    out_t = pl.pallas_call(
        _tc_pool_kernel,
        grid=(nw,),
        in_specs=[
            pl.BlockSpec((1, L, 1, BW), lambda i: (i, 0, 0, 0)),
            pl.BlockSpec((1, 1, BW), lambda i: (i, 0, 0)),
            pl.BlockSpec((L, D, BW), lambda i: (0, 0, i)),
        ],
        out_specs=pl.BlockSpec((D, BW), lambda i: (0, i)),
        out_shape=jax.ShapeDtypeStruct((D, B), jnp.float32),
    )(a4, inv3, xt)
    return out_t.T
